# Initial kernel scaffold; baseline (speedup 1.0000x reference)
import jax
import jax.numpy as jnp
from jax.experimental import pallas as pl


def kernel(x, edge_index, edge_weight, W1, b1, W2, b2, W3, b3, Wc, bc):
    raise NotImplementedError("write your pallas kernel here")


# speedup vs baseline: 151.0614x; 151.0614x over previous
import functools

import jax
import jax.numpy as jnp
from jax import lax
from jax.experimental import pallas as pl
from jax.experimental.pallas import tpu as pltpu
from jax.experimental.pallas import tpu_sc as plsc

_LANES = 128
_NSC = 2
_NTILE = 16


def _sc_mesh():
    return plsc.VectorSubcoreMesh(core_axis_name="c", subcore_axis_name="s")


def _deg_body(edge_ref, ew_ref, out_ref, deg_sp, dst_v, ew_v, stage_v,
              *, chunk, nchunk, ept, rpt, e):
    cid = lax.axis_index("c")
    sid = lax.axis_index("s")
    wid = cid * _NTILE + sid
    zv = jnp.zeros((16,), jnp.float32)

    def zstep(g, c2):
        stage_v[pl.ds(g * 16, 16)] = zv
        return c2

    lax.fori_loop(0, rpt // 16, zstep, 0)
    pltpu.sync_copy(stage_v, deg_sp.at[pl.ds(sid * rpt, rpt)])
    plsc.subcore_barrier()
    base0 = wid * ept

    def step(i, carry):
        base = base0 + i * chunk
        pltpu.sync_copy(edge_ref.at[pl.ds(e + base, chunk)], dst_v)
        pltpu.sync_copy(ew_ref.at[pl.ds(base, chunk)], ew_v)
        pltpu.sync_copy(ew_v, deg_sp.at[dst_v], add=True)
        return carry

    lax.fori_loop(0, nchunk, step, 0)
    plsc.subcore_barrier()
    pltpu.sync_copy(deg_sp.at[pl.ds(sid * rpt, rpt)], stage_v)
    pltpu.sync_copy(stage_v,
                    out_ref.at[pl.ds(cid * (rpt * _NTILE) + sid * rpt, rpt)])


def _edge_body(edge_ref, ew_ref, t0_ref, t1_ref, out0_ref, out1_ref,
               t0_sp, t1_sp, a0_sp, a1_sp, src_v, dst_v, ew_v, r0_v, r1_v,
               stage_v,
               *, chunk, nchunk, ept, rpt, e):
    cid = lax.axis_index("c")
    sid = lax.axis_index("s")
    wid = cid * _NTILE + sid
    sl = pl.ds(sid * rpt, rpt)

    zv = jnp.zeros((16,), jnp.float32)

    def zstep(g, c2):
        stage_v[pl.ds(g * 16, 16)] = zv
        return c2

    lax.fori_loop(0, rpt // 16, zstep, 0)
    pltpu.sync_copy(stage_v, a0_sp.at[sl])
    pltpu.sync_copy(stage_v, a1_sp.at[sl])
    pltpu.sync_copy(t0_ref.at[sl], stage_v)
    pltpu.sync_copy(stage_v, t0_sp.at[sl])
    pltpu.sync_copy(t1_ref.at[sl], stage_v)
    pltpu.sync_copy(stage_v, t1_sp.at[sl])
    plsc.subcore_barrier()

    base0 = wid * ept

    def step(i, carry):
        base = base0 + i * chunk
        pltpu.sync_copy(edge_ref.at[pl.ds(base, chunk)], src_v)
        pltpu.sync_copy(edge_ref.at[pl.ds(e + base, chunk)], dst_v)
        pltpu.sync_copy(ew_ref.at[pl.ds(base, chunk)], ew_v)
        pltpu.sync_copy(t0_sp.at[src_v], r0_v)
        pltpu.sync_copy(t1_sp.at[src_v], r1_v)

        def mul(g, c2):
            o = pl.ds(g * 16, 16)
            w = ew_v[o]
            r0_v[o] = r0_v[o] * w
            r1_v[o] = r1_v[o] * w
            return c2

        lax.fori_loop(0, chunk // 16, mul, 0)
        pltpu.sync_copy(r0_v, a0_sp.at[dst_v], add=True)
        pltpu.sync_copy(r1_v, a1_sp.at[dst_v], add=True)
        return carry

    lax.fori_loop(0, nchunk, step, 0)
    plsc.subcore_barrier()
    osl = pl.ds(cid * (rpt * _NTILE) + sid * rpt, rpt)
    pltpu.sync_copy(a0_sp.at[sl], stage_v)
    pltpu.sync_copy(stage_v, out0_ref.at[osl])
    pltpu.sync_copy(a1_sp.at[sl], stage_v)
    pltpu.sync_copy(stage_v, out1_ref.at[osl])


def _deg_kernel(edge_index, edge_weight, *, npad, chunk, nchunk, ept, e):
    rpt = npad // _NTILE
    body = functools.partial(_deg_body, chunk=chunk, nchunk=nchunk,
                             ept=ept, rpt=rpt, e=e)
    return pl.kernel(
        body,
        out_type=jax.ShapeDtypeStruct((_NSC * npad,), jnp.float32),
        mesh=_sc_mesh(),
        scratch_types=[
            pltpu.VMEM_SHARED((npad,), jnp.float32),
            pltpu.VMEM((chunk,), jnp.int32),
            pltpu.VMEM((chunk,), jnp.float32),
            pltpu.VMEM((rpt,), jnp.float32),
        ],
    )(edge_index, edge_weight)


def _edge_kernel(edge_index, edge_weight, t0, t1, *, npad, chunk,
                 nchunk, ept, e):
    rpt = npad // _NTILE
    body = functools.partial(_edge_body, chunk=chunk, nchunk=nchunk,
                             ept=ept, rpt=rpt, e=e)
    shp = jax.ShapeDtypeStruct((_NSC * npad,), jnp.float32)
    return pl.kernel(
        body,
        out_type=(shp, shp),
        mesh=_sc_mesh(),
        scratch_types=[
            pltpu.VMEM_SHARED((npad,), jnp.float32),
            pltpu.VMEM_SHARED((npad,), jnp.float32),
            pltpu.VMEM_SHARED((npad,), jnp.float32),
            pltpu.VMEM_SHARED((npad,), jnp.float32),
            pltpu.VMEM((chunk,), jnp.int32),
            pltpu.VMEM((chunk,), jnp.int32),
            pltpu.VMEM((chunk,), jnp.float32),
            pltpu.VMEM((chunk,), jnp.float32),
            pltpu.VMEM((chunk,), jnp.float32),
            pltpu.VMEM((rpt,), jnp.float32),
        ],
    )(edge_index, edge_weight, t0, t1)


def _dense_a_body(degp_ref, x0_ref, x1_ref, w_ref, dinv_ref, t0_ref, t1_ref):
    deg = degp_ref[0] + degp_ref[1] + 1.0
    dinv = lax.rsqrt(deg)
    dinv_ref[...] = dinv
    x0 = x0_ref[...]
    x1 = x1_ref[...]
    t0_ref[...] = dinv * (x0 * w_ref[0, 0] + x1 * w_ref[1, 0])
    t1_ref[...] = dinv * (x0 * w_ref[0, 1] + x1 * w_ref[1, 1])


def _dense_mid_body(pc0_ref, pc1_ref, t0_ref, t1_ref, dinv_ref, b_ref,
                    wn_ref, h0_ref, h1_ref, u0_ref, u1_ref):
    dinv = dinv_ref[...]
    s0 = dinv * (pc0_ref[0] + pc0_ref[1] + t0_ref[...]) + b_ref[0]
    s1 = dinv * (pc1_ref[0] + pc1_ref[1] + t1_ref[...]) + b_ref[1]
    h0 = jnp.tanh(s0)
    h1 = jnp.tanh(s1)
    h0_ref[...] = h0
    h1_ref[...] = h1
    u0_ref[...] = dinv * (h0 * wn_ref[0, 0] + h1 * wn_ref[1, 0])
    u1_ref[...] = dinv * (h0 * wn_ref[0, 1] + h1 * wn_ref[1, 1])


def _dense_fin_body(pc0_ref, pc1_ref, t0_ref, t1_ref, dinv_ref, b_ref,
                    wc_ref, bc_ref, h0_ref, h1_ref, lg_ref):
    dinv = dinv_ref[...]
    s0 = dinv * (pc0_ref[0] + pc0_ref[1] + t0_ref[...]) + b_ref[0]
    s1 = dinv * (pc1_ref[0] + pc1_ref[1] + t1_ref[...]) + b_ref[1]
    h0 = jnp.tanh(s0)
    h1 = jnp.tanh(s1)
    h0_ref[...] = h0
    h1_ref[...] = h1
    lg_ref[...] = h0 * wc_ref[0, 0] + h1 * wc_ref[1, 0] + bc_ref[0]


def _vspec():
    return pl.BlockSpec(memory_space=pltpu.VMEM)


def _sspec():
    return pl.BlockSpec(memory_space=pltpu.SMEM)


def _dense_a(degp, x0, x1, w, *, rows):
    shp = jax.ShapeDtypeStruct((rows, _LANES), jnp.float32)
    return pl.pallas_call(
        _dense_a_body,
        out_shape=(shp, shp, shp),
        in_specs=[_vspec(), _vspec(), _vspec(), _sspec()],
        out_specs=(_vspec(), _vspec(), _vspec()),
    )(degp, x0, x1, w)


def _dense_mid(pc0, pc1, t0, t1, dinv, b, wn, *, rows):
    shp = jax.ShapeDtypeStruct((rows, _LANES), jnp.float32)
    return pl.pallas_call(
        _dense_mid_body,
        out_shape=(shp, shp, shp, shp),
        in_specs=[_vspec()] * 5 + [_sspec(), _sspec()],
        out_specs=(_vspec(),) * 4,
    )(pc0, pc1, t0, t1, dinv, b, wn)


def _dense_fin(pc0, pc1, t0, t1, dinv, b, wc, bc, *, rows):
    shp = jax.ShapeDtypeStruct((rows, _LANES), jnp.float32)
    return pl.pallas_call(
        _dense_fin_body,
        out_shape=(shp, shp, shp),
        in_specs=[_vspec()] * 5 + [_sspec(), _sspec(), _sspec()],
        out_specs=(_vspec(),) * 3,
    )(pc0, pc1, t0, t1, dinv, b, wc, bc)


def kernel(x, edge_index, edge_weight, W1, b1, W2, b2, W3, b3, Wc, bc):
    n = x.shape[0]
    e = edge_index.shape[1]
    npad = ((n + _LANES - 1) // _LANES) * _LANES
    rows = npad // _LANES
    nworker = _NSC * _NTILE
    ept = e // nworker
    chunk = 10000
    while ept % chunk:
        chunk //= 2
    nchunk = ept // chunk

    edge_flat = edge_index.astype(jnp.int32).reshape(-1)
    ew = edge_weight.astype(jnp.float32)

    xp = jnp.pad(x.astype(jnp.float32), ((0, npad - n), (0, 0)))
    x0 = xp[:, 0].reshape(rows, _LANES)
    x1 = xp[:, 1].reshape(rows, _LANES)
    degp = _deg_kernel(edge_flat, ew, npad=npad, chunk=chunk,
                       nchunk=nchunk, ept=ept, e=e)
    degp = degp.reshape(_NSC, rows, _LANES)

    dinv, t0, t1 = _dense_a(degp, x0, x1, W1.astype(jnp.float32), rows=rows)

    def edge_pass(t0c, t1c):
        p0, p1 = _edge_kernel(edge_flat, ew, t0c.reshape(-1),
                              t1c.reshape(-1), npad=npad,
                              chunk=chunk, nchunk=nchunk, ept=ept, e=e)
        pc0 = p0.reshape(_NSC, rows, _LANES)
        pc1 = p1.reshape(_NSC, rows, _LANES)
        return pc0, pc1

    pc0, pc1 = edge_pass(t0, t1)
    h0, h1, t0, t1 = _dense_mid(pc0, pc1, t0, t1, dinv,
                                b1.astype(jnp.float32),
                                W2.astype(jnp.float32), rows=rows)
    pc0, pc1 = edge_pass(t0, t1)
    h0, h1, t0, t1 = _dense_mid(pc0, pc1, t0, t1, dinv,
                                b2.astype(jnp.float32),
                                W3.astype(jnp.float32), rows=rows)
    pc0, pc1 = edge_pass(t0, t1)
    h0, h1, lg = _dense_fin(pc0, pc1, t0, t1, dinv,
                            b3.astype(jnp.float32),
                            Wc.astype(jnp.float32),
                            bc.astype(jnp.float32), rows=rows)

    h = jnp.stack([h0.reshape(-1)[:n], h1.reshape(-1)[:n]], axis=-1)
    logits = lg.reshape(-1)[:n][:, None]
    return (logits, h)


# speedup vs baseline: 175.0982x; 1.1591x over previous
import functools

import jax
import jax.numpy as jnp
from jax import lax
from jax.experimental import pallas as pl
from jax.experimental.pallas import tpu as pltpu
from jax.experimental.pallas import tpu_sc as plsc

_LANES = 128
_NSC = 2
_NTILE = 16


def _sc_mesh():
    return plsc.VectorSubcoreMesh(core_axis_name="c", subcore_axis_name="s")


def _deg_body(edge_ref, ew_ref, out_ref, deg_sp, dst_v, ew_v, stage_v,
              *, chunk, nchunk, ept, rpt, e):
    cid = lax.axis_index("c")
    sid = lax.axis_index("s")
    wid = cid * _NTILE + sid
    zv = jnp.zeros((16,), jnp.float32)

    def zstep(g, c2):
        stage_v[pl.ds(g * 16, 16)] = zv
        return c2

    lax.fori_loop(0, rpt // 16, zstep, 0)
    pltpu.sync_copy(stage_v, deg_sp.at[pl.ds(sid * rpt, rpt)])
    plsc.subcore_barrier()
    base0 = wid * ept

    def step(i, carry):
        base = base0 + i * chunk
        pltpu.sync_copy(edge_ref.at[pl.ds(e + base, chunk)], dst_v)
        pltpu.sync_copy(ew_ref.at[pl.ds(base, chunk)], ew_v)
        pltpu.sync_copy(ew_v, deg_sp.at[dst_v], add=True)
        return carry

    lax.fori_loop(0, nchunk, step, 0)
    plsc.subcore_barrier()
    pltpu.sync_copy(deg_sp.at[pl.ds(sid * rpt, rpt)], stage_v)
    pltpu.sync_copy(stage_v,
                    out_ref.at[pl.ds(cid * (rpt * _NTILE) + sid * rpt, rpt)])


def _edge_body(edge_ref, ew_ref, t0_ref, t1_ref, out0_ref, out1_ref,
               t0_sp, t1_sp, a0_sp, a1_sp,
               src_a, src_b, src_c, dst_a, dst_b, dst_c, ew_a, ew_b, ew_c,
               r0_a, r0_b, r1_a, r1_b, stage_v,
               sem_i0, sem_i1, sem_i2, sem_g0, sem_g1, sem_s0, sem_s1,
               *, chunk, nchunk, ept, rpt, e):
    cid = lax.axis_index("c")
    sid = lax.axis_index("s")
    wid = cid * _NTILE + sid
    sl = pl.ds(sid * rpt, rpt)
    srcs = [src_a, src_b, src_c]
    dsts = [dst_a, dst_b, dst_c]
    ews = [ew_a, ew_b, ew_c]
    sem_is = [sem_i0, sem_i1, sem_i2]
    r0s = [r0_a, r0_b]
    r1s = [r1_a, r1_b]
    sem_gs = [sem_g0, sem_g1]
    sem_ss = [sem_s0, sem_s1]
    base0 = wid * ept

    def issue_in(i):
        s3 = i % 3
        base = base0 + i * chunk
        return [
            pltpu.async_copy(edge_ref.at[pl.ds(base, chunk)], srcs[s3],
                             sem_is[s3]),
            pltpu.async_copy(edge_ref.at[pl.ds(e + base, chunk)], dsts[s3],
                             sem_is[s3]),
            pltpu.async_copy(ew_ref.at[pl.ds(base, chunk)], ews[s3],
                             sem_is[s3]),
        ]

    pend_in = {0: issue_in(0), 1: issue_in(1)}

    zv = jnp.zeros((16,), jnp.float32)

    def zstep(g, c2):
        stage_v[pl.ds(g * 16, 16)] = zv
        return c2

    lax.fori_loop(0, rpt // 16, zstep, 0)
    pltpu.sync_copy(stage_v, a0_sp.at[sl])
    pltpu.sync_copy(stage_v, a1_sp.at[sl])
    pltpu.sync_copy(t0_ref.at[sl], stage_v)
    pltpu.sync_copy(stage_v, t0_sp.at[sl])
    pltpu.sync_copy(t1_ref.at[sl], stage_v)
    pltpu.sync_copy(stage_v, t1_sp.at[sl])
    plsc.subcore_barrier()

    pend_sc = {}
    for i in range(nchunk):
        s3 = i % 3
        s2 = i % 2
        for cp in pend_in.pop(i):
            cp.wait()
        g0 = pltpu.async_copy(t0_sp.at[srcs[s3]], r0s[s2], sem_gs[s2])
        g1 = pltpu.async_copy(t1_sp.at[srcs[s3]], r1s[s2], sem_gs[s2])
        g0.wait()
        g1.wait()
        r0_v = r0s[s2]
        r1_v = r1s[s2]
        ew_v = ews[s3]

        def mul(g, c2):
            o = pl.ds(g * 16, 16)
            w = ew_v[o]
            r0_v[o] = r0_v[o] * w
            r1_v[o] = r1_v[o] * w
            return c2

        lax.fori_loop(0, chunk // 16, mul, 0)
        if i >= 1:
            for cp in pend_sc.pop(i - 1, []):
                cp.wait()
        pend_sc[i] = [
            pltpu.async_copy(r0_v, a0_sp.at[dsts[s3]], sem_ss[s2], add=True),
            pltpu.async_copy(r1_v, a1_sp.at[dsts[s3]], sem_ss[s2], add=True),
        ]
        if i + 2 < nchunk:
            pend_in[i + 2] = issue_in(i + 2)
    for pend in pend_sc.values():
        for cp in pend:
            cp.wait()
    plsc.subcore_barrier()
    osl = pl.ds(cid * (rpt * _NTILE) + sid * rpt, rpt)
    pltpu.sync_copy(a0_sp.at[sl], stage_v)
    pltpu.sync_copy(stage_v, out0_ref.at[osl])
    pltpu.sync_copy(a1_sp.at[sl], stage_v)
    pltpu.sync_copy(stage_v, out1_ref.at[osl])


def _deg_kernel(edge_index, edge_weight, *, npad, chunk, nchunk, ept, e):
    rpt = npad // _NTILE
    body = functools.partial(_deg_body, chunk=chunk, nchunk=nchunk,
                             ept=ept, rpt=rpt, e=e)
    return pl.kernel(
        body,
        out_type=jax.ShapeDtypeStruct((_NSC * npad,), jnp.float32),
        mesh=_sc_mesh(),
        scratch_types=[
            pltpu.VMEM_SHARED((npad,), jnp.float32),
            pltpu.VMEM((chunk,), jnp.int32),
            pltpu.VMEM((chunk,), jnp.float32),
            pltpu.VMEM((rpt,), jnp.float32),
        ],
    )(edge_index, edge_weight)


def _edge_kernel(edge_index, edge_weight, t0, t1, *, npad, chunk,
                 nchunk, ept, e):
    rpt = npad // _NTILE
    body = functools.partial(_edge_body, chunk=chunk, nchunk=nchunk,
                             ept=ept, rpt=rpt, e=e)
    shp = jax.ShapeDtypeStruct((_NSC * npad,), jnp.float32)
    return pl.kernel(
        body,
        out_type=(shp, shp),
        mesh=_sc_mesh(),
        scratch_types=(
            [pltpu.VMEM_SHARED((npad,), jnp.float32)] * 4
            + [pltpu.VMEM((chunk,), jnp.int32)] * 6
            + [pltpu.VMEM((chunk,), jnp.float32)] * 3
            + [pltpu.VMEM((chunk,), jnp.float32)] * 4
            + [pltpu.VMEM((rpt,), jnp.float32)]
            + [pltpu.SemaphoreType.DMA] * 7
        ),
    )(edge_index, edge_weight, t0, t1)


def _dense_a_body(degp_ref, x0_ref, x1_ref, w_ref, dinv_ref, t0_ref, t1_ref):
    deg = degp_ref[0] + degp_ref[1] + 1.0
    dinv = lax.rsqrt(deg)
    dinv_ref[...] = dinv
    x0 = x0_ref[...]
    x1 = x1_ref[...]
    t0_ref[...] = dinv * (x0 * w_ref[0, 0] + x1 * w_ref[1, 0])
    t1_ref[...] = dinv * (x0 * w_ref[0, 1] + x1 * w_ref[1, 1])


def _dense_mid_body(pc0_ref, pc1_ref, t0_ref, t1_ref, dinv_ref, b_ref,
                    wn_ref, h0_ref, h1_ref, u0_ref, u1_ref):
    dinv = dinv_ref[...]
    s0 = dinv * (pc0_ref[0] + pc0_ref[1] + t0_ref[...]) + b_ref[0]
    s1 = dinv * (pc1_ref[0] + pc1_ref[1] + t1_ref[...]) + b_ref[1]
    h0 = jnp.tanh(s0)
    h1 = jnp.tanh(s1)
    h0_ref[...] = h0
    h1_ref[...] = h1
    u0_ref[...] = dinv * (h0 * wn_ref[0, 0] + h1 * wn_ref[1, 0])
    u1_ref[...] = dinv * (h0 * wn_ref[0, 1] + h1 * wn_ref[1, 1])


def _dense_fin_body(pc0_ref, pc1_ref, t0_ref, t1_ref, dinv_ref, b_ref,
                    wc_ref, bc_ref, h0_ref, h1_ref, lg_ref):
    dinv = dinv_ref[...]
    s0 = dinv * (pc0_ref[0] + pc0_ref[1] + t0_ref[...]) + b_ref[0]
    s1 = dinv * (pc1_ref[0] + pc1_ref[1] + t1_ref[...]) + b_ref[1]
    h0 = jnp.tanh(s0)
    h1 = jnp.tanh(s1)
    h0_ref[...] = h0
    h1_ref[...] = h1
    lg_ref[...] = h0 * wc_ref[0, 0] + h1 * wc_ref[1, 0] + bc_ref[0]


def _vspec():
    return pl.BlockSpec(memory_space=pltpu.VMEM)


def _sspec():
    return pl.BlockSpec(memory_space=pltpu.SMEM)


def _dense_a(degp, x0, x1, w, *, rows):
    shp = jax.ShapeDtypeStruct((rows, _LANES), jnp.float32)
    return pl.pallas_call(
        _dense_a_body,
        out_shape=(shp, shp, shp),
        in_specs=[_vspec(), _vspec(), _vspec(), _sspec()],
        out_specs=(_vspec(), _vspec(), _vspec()),
    )(degp, x0, x1, w)


def _dense_mid(pc0, pc1, t0, t1, dinv, b, wn, *, rows):
    shp = jax.ShapeDtypeStruct((rows, _LANES), jnp.float32)
    return pl.pallas_call(
        _dense_mid_body,
        out_shape=(shp, shp, shp, shp),
        in_specs=[_vspec()] * 5 + [_sspec(), _sspec()],
        out_specs=(_vspec(),) * 4,
    )(pc0, pc1, t0, t1, dinv, b, wn)


def _dense_fin(pc0, pc1, t0, t1, dinv, b, wc, bc, *, rows):
    shp = jax.ShapeDtypeStruct((rows, _LANES), jnp.float32)
    return pl.pallas_call(
        _dense_fin_body,
        out_shape=(shp, shp, shp),
        in_specs=[_vspec()] * 5 + [_sspec(), _sspec(), _sspec()],
        out_specs=(_vspec(),) * 3,
    )(pc0, pc1, t0, t1, dinv, b, wc, bc)


def kernel(x, edge_index, edge_weight, W1, b1, W2, b2, W3, b3, Wc, bc):
    n = x.shape[0]
    e = edge_index.shape[1]
    npad = ((n + _LANES - 1) // _LANES) * _LANES
    rows = npad // _LANES
    nworker = _NSC * _NTILE
    ept = e // nworker
    chunk = 4000
    while ept % chunk or chunk % 16:
        chunk //= 2
    nchunk = ept // chunk

    edge_flat = edge_index.astype(jnp.int32).reshape(-1)
    ew = edge_weight.astype(jnp.float32)

    xp = jnp.pad(x.astype(jnp.float32), ((0, npad - n), (0, 0)))
    x0 = xp[:, 0].reshape(rows, _LANES)
    x1 = xp[:, 1].reshape(rows, _LANES)
    degp = _deg_kernel(edge_flat, ew, npad=npad, chunk=chunk,
                       nchunk=nchunk, ept=ept, e=e)
    degp = degp.reshape(_NSC, rows, _LANES)

    dinv, t0, t1 = _dense_a(degp, x0, x1, W1.astype(jnp.float32), rows=rows)

    def edge_pass(t0c, t1c):
        p0, p1 = _edge_kernel(edge_flat, ew, t0c.reshape(-1),
                              t1c.reshape(-1), npad=npad,
                              chunk=chunk, nchunk=nchunk, ept=ept, e=e)
        pc0 = p0.reshape(_NSC, rows, _LANES)
        pc1 = p1.reshape(_NSC, rows, _LANES)
        return pc0, pc1

    pc0, pc1 = edge_pass(t0, t1)
    h0, h1, t0, t1 = _dense_mid(pc0, pc1, t0, t1, dinv,
                                b1.astype(jnp.float32),
                                W2.astype(jnp.float32), rows=rows)
    pc0, pc1 = edge_pass(t0, t1)
    h0, h1, t0, t1 = _dense_mid(pc0, pc1, t0, t1, dinv,
                                b2.astype(jnp.float32),
                                W3.astype(jnp.float32), rows=rows)
    pc0, pc1 = edge_pass(t0, t1)
    h0, h1, lg = _dense_fin(pc0, pc1, t0, t1, dinv,
                            b3.astype(jnp.float32),
                            Wc.astype(jnp.float32),
                            bc.astype(jnp.float32), rows=rows)

    h = jnp.stack([h0.reshape(-1)[:n], h1.reshape(-1)[:n]], axis=-1)
    logits = lg.reshape(-1)[:n][:, None]
    return (logits, h)
